# baseline (device time: 20394 ns/iter reference)
import jax
import jax.numpy as jnp
from jax import lax
from jax.experimental import pallas as pl
from jax.experimental.pallas import tpu as pltpu

N_DEV = 8
B, SQ, SKV = 2, 128, 128
HQ_PER, DH = 4, 64
D_MODEL = 512
D_SHARD = HQ_PER * DH
ROWS = B * SQ
CHUNK = ROWS // N_DEV


def kernel(x, Wq, K_ext, V_ext, Wo):
    my = lax.axis_index("i")
    k_sl = lax.dynamic_slice_in_dim(K_ext, my * HQ_PER, HQ_PER, axis=2)
    v_sl = lax.dynamic_slice_in_dim(V_ext, my * HQ_PER, HQ_PER, axis=2)
    k_t = jnp.transpose(k_sl, (0, 2, 1, 3))
    v_t = jnp.transpose(v_sl, (0, 2, 1, 3))
    x2 = x.reshape(ROWS, D_MODEL)

    def body(x_ref, wq_ref, k_ref, v_ref, wo_ref, out_ref,
             ctx_buf, partial_buf, recv_buf, ag_buf,
             rs_send, rs_recv, ag_send, ag_recv):
        my_pos = lax.axis_index("i")

        barrier = pltpu.get_barrier_semaphore()
        for d in range(1, N_DEV):
            peer = lax.rem(my_pos + d, N_DEV)
            pl.semaphore_signal(
                barrier, inc=1,
                device_id=(peer,), device_id_type=pl.DeviceIdType.MESH,
            )
        pl.semaphore_wait(barrier, N_DEV - 1)

        xb = x_ref[:].astype(jnp.bfloat16)
        wqb = wq_ref[:].astype(jnp.bfloat16)
        q = lax.dot(xb, wqb, preferred_element_type=jnp.float32) * 0.125
        for b in range(B):
            for h in range(HQ_PER):
                qbh = q[b * SQ:(b + 1) * SQ, h * DH:(h + 1) * DH]
                qbh = qbh.astype(jnp.bfloat16)
                kbh = k_ref[b, h].astype(jnp.bfloat16)
                s = lax.dot_general(
                    qbh, kbh, (((1,), (1,)), ((), ())),
                    preferred_element_type=jnp.float32,
                )
                s = s - jnp.max(s, axis=-1, keepdims=True)
                w = jnp.exp(s)
                w = w / jnp.sum(w, axis=-1, keepdims=True)
                vbh = v_ref[b, h].astype(jnp.bfloat16)
                cbh = lax.dot(w.astype(jnp.bfloat16), vbh,
                              preferred_element_type=jnp.float32)
                ctx_buf[b * SQ:(b + 1) * SQ, h * DH:(h + 1) * DH] = cbh
        partial = lax.dot(ctx_buf[:].astype(jnp.bfloat16),
                          wo_ref[:].astype(jnp.bfloat16),
                          preferred_element_type=jnp.float32)
        partial_buf[:] = partial.reshape(N_DEV, CHUNK, D_MODEL)
        recv_buf[pl.ds(0, 1)] = partial_buf[pl.ds(my_pos, 1)]

        rs_ops = []
        for d in range(1, N_DEV):
            t = lax.rem(my_pos + d, N_DEV)
            op = pltpu.make_async_remote_copy(
                src_ref=partial_buf.at[pl.ds(t, 1)],
                dst_ref=recv_buf.at[pl.ds(d, 1)],
                send_sem=rs_send.at[d],
                recv_sem=rs_recv.at[d],
                device_id=(t,),
                device_id_type=pl.DeviceIdType.MESH,
            )
            op.start()
            rs_ops.append(op)
        for op in rs_ops:
            op.wait_recv()
        reduced = jnp.sum(recv_buf[:], axis=0)

        ag_buf[pl.ds(0, 1)] = reduced[None]
        ag_ops = []
        for d in range(1, N_DEV):
            t = lax.rem(my_pos + d, N_DEV)
            op = pltpu.make_async_remote_copy(
                src_ref=ag_buf.at[pl.ds(0, 1)],
                dst_ref=ag_buf.at[pl.ds(d, 1)],
                send_sem=ag_send.at[d],
                recv_sem=ag_recv.at[d],
                device_id=(t,),
                device_id_type=pl.DeviceIdType.MESH,
            )
            op.start()
            ag_ops.append(op)

        out_ref[pl.ds(my_pos, 1)] = ag_buf[pl.ds(0, 1)]
        for d in range(1, N_DEV):
            ag_ops[d - 1].wait_recv()
            origin = lax.rem(my_pos + (N_DEV - d), N_DEV)
            out_ref[pl.ds(origin, 1)] = ag_buf[pl.ds(d, 1)]

        for op in rs_ops:
            op.wait_send()
        for op in ag_ops:
            op.wait_send()

    out = pl.pallas_call(
        body,
        out_shape=jax.ShapeDtypeStruct((N_DEV, CHUNK, D_MODEL), jnp.float32),
        in_specs=[pl.BlockSpec(memory_space=pltpu.VMEM)] * 5,
        out_specs=pl.BlockSpec(memory_space=pltpu.VMEM),
        scratch_shapes=[
            pltpu.VMEM((ROWS, D_SHARD), jnp.float32),
            pltpu.VMEM((N_DEV, CHUNK, D_MODEL), jnp.float32),
            pltpu.VMEM((N_DEV, CHUNK, D_MODEL), jnp.float32),
            pltpu.VMEM((N_DEV, CHUNK, D_MODEL), jnp.float32),
            pltpu.SemaphoreType.DMA((N_DEV,)),
            pltpu.SemaphoreType.DMA((N_DEV,)),
            pltpu.SemaphoreType.DMA((N_DEV,)),
            pltpu.SemaphoreType.DMA((N_DEV,)),
        ],
        compiler_params=pltpu.CompilerParams(collective_id=0),
    )(x2, Wq, k_t, v_t, Wo)
    return out.reshape(B, SQ, D_MODEL)


# device time: 17740 ns/iter; 1.1496x vs baseline; 1.1496x over previous
import jax
import jax.numpy as jnp
from jax import lax
from jax.experimental import pallas as pl
from jax.experimental.pallas import tpu as pltpu

N_DEV = 8
B, SQ, SKV = 2, 128, 128
HQ_PER, DH = 4, 64
D_MODEL = 512
D_SHARD = HQ_PER * DH
ROWS = B * SQ
CHUNK = ROWS // N_DEV


def kernel(x, Wq, K_ext, V_ext, Wo):
    my = lax.axis_index("i")
    k_sl = lax.dynamic_slice_in_dim(K_ext, my * HQ_PER, HQ_PER, axis=2)
    v_sl = lax.dynamic_slice_in_dim(V_ext, my * HQ_PER, HQ_PER, axis=2)
    k_t = jnp.transpose(k_sl, (0, 2, 1, 3))
    v_t = jnp.transpose(v_sl, (0, 2, 1, 3))
    x2 = x.reshape(ROWS, D_MODEL)

    def body(x_ref, wq_ref, k_ref, v_ref, wo_ref, out_ref,
             ctx_buf, partial_buf, recv_buf, ag_buf,
             rs_send, rs_recv, ag_send, ag_recv):
        my_pos = lax.axis_index("i")

        barrier = pltpu.get_barrier_semaphore()
        for d in range(1, N_DEV):
            peer = lax.rem(my_pos + d, N_DEV)
            pl.semaphore_signal(
                barrier, inc=1,
                device_id=(peer,), device_id_type=pl.DeviceIdType.MESH,
            )
        pl.semaphore_wait(barrier, N_DEV - 1)

        xb = x_ref[:].astype(jnp.bfloat16)
        wqb = wq_ref[:].astype(jnp.bfloat16)
        q = lax.dot(xb, wqb, preferred_element_type=jnp.float32) * 0.125
        for b in range(B):
            for h in range(HQ_PER):
                qbh = q[b * SQ:(b + 1) * SQ, h * DH:(h + 1) * DH]
                qbh = qbh.astype(jnp.bfloat16)
                kbh = k_ref[b, h].astype(jnp.bfloat16)
                s = lax.dot_general(
                    qbh, kbh, (((1,), (1,)), ((), ())),
                    preferred_element_type=jnp.float32,
                )
                s = s - jnp.max(s, axis=-1, keepdims=True)
                w = jnp.exp(s)
                w = w / jnp.sum(w, axis=-1, keepdims=True)
                vbh = v_ref[b, h].astype(jnp.bfloat16)
                cbh = lax.dot(w.astype(jnp.bfloat16), vbh,
                              preferred_element_type=jnp.float32)
                ctx_buf[b * SQ:(b + 1) * SQ, h * DH:(h + 1) * DH] = cbh
        partial = lax.dot(ctx_buf[:].astype(jnp.bfloat16),
                          wo_ref[:].astype(jnp.bfloat16),
                          preferred_element_type=jnp.float32)
        partial_buf[:] = partial.astype(jnp.bfloat16).reshape(
            N_DEV, CHUNK, D_MODEL)
        recv_buf[pl.ds(0, 1)] = partial_buf[pl.ds(my_pos, 1)]

        rs_ops = []
        for d in range(1, N_DEV):
            t = lax.rem(my_pos + d, N_DEV)
            op = pltpu.make_async_remote_copy(
                src_ref=partial_buf.at[pl.ds(t, 1)],
                dst_ref=recv_buf.at[pl.ds(d, 1)],
                send_sem=rs_send.at[d],
                recv_sem=rs_recv.at[d],
                device_id=(t,),
                device_id_type=pl.DeviceIdType.MESH,
            )
            op.start()
            rs_ops.append(op)
        for op in rs_ops:
            op.wait_recv()
        reduced = jnp.sum(recv_buf[:].astype(jnp.float32), axis=0)

        ag_buf[pl.ds(0, 1)] = reduced.astype(jnp.bfloat16)[None]
        ag_ops = []
        for d in range(1, N_DEV):
            t = lax.rem(my_pos + d, N_DEV)
            op = pltpu.make_async_remote_copy(
                src_ref=ag_buf.at[pl.ds(0, 1)],
                dst_ref=ag_buf.at[pl.ds(d, 1)],
                send_sem=ag_send.at[d],
                recv_sem=ag_recv.at[d],
                device_id=(t,),
                device_id_type=pl.DeviceIdType.MESH,
            )
            op.start()
            ag_ops.append(op)

        out_ref[pl.ds(my_pos, 1)] = ag_buf[pl.ds(0, 1)].astype(jnp.float32)
        for d in range(1, N_DEV):
            ag_ops[d - 1].wait_recv()
            origin = lax.rem(my_pos + (N_DEV - d), N_DEV)
            out_ref[pl.ds(origin, 1)] = ag_buf[pl.ds(d, 1)].astype(jnp.float32)

        for op in rs_ops:
            op.wait_send()
        for op in ag_ops:
            op.wait_send()

    out = pl.pallas_call(
        body,
        out_shape=jax.ShapeDtypeStruct((N_DEV, CHUNK, D_MODEL), jnp.float32),
        in_specs=[pl.BlockSpec(memory_space=pltpu.VMEM)] * 5,
        out_specs=pl.BlockSpec(memory_space=pltpu.VMEM),
        scratch_shapes=[
            pltpu.VMEM((ROWS, D_SHARD), jnp.float32),
            pltpu.VMEM((N_DEV, CHUNK, D_MODEL), jnp.bfloat16),
            pltpu.VMEM((N_DEV, CHUNK, D_MODEL), jnp.bfloat16),
            pltpu.VMEM((N_DEV, CHUNK, D_MODEL), jnp.bfloat16),
            pltpu.SemaphoreType.DMA((N_DEV,)),
            pltpu.SemaphoreType.DMA((N_DEV,)),
            pltpu.SemaphoreType.DMA((N_DEV,)),
            pltpu.SemaphoreType.DMA((N_DEV,)),
        ],
        compiler_params=pltpu.CompilerParams(collective_id=0),
    )(x2, Wq, k_t, v_t, Wo)
    return out.reshape(B, SQ, D_MODEL)


# device time: 15806 ns/iter; 1.2903x vs baseline; 1.1224x over previous
import jax
import jax.numpy as jnp
from jax import lax
from jax.experimental import pallas as pl
from jax.experimental.pallas import tpu as pltpu

N_DEV = 8
B, SQ, SKV = 2, 128, 128
HQ_PER, DH = 4, 64
D_MODEL = 512
D_SHARD = HQ_PER * DH
ROWS = B * SQ
CHUNK = ROWS // N_DEV


def kernel(x, Wq, K_ext, V_ext, Wo):
    my = lax.axis_index("i")
    k_sl = lax.dynamic_slice_in_dim(K_ext, my * HQ_PER, HQ_PER, axis=2)
    v_sl = lax.dynamic_slice_in_dim(V_ext, my * HQ_PER, HQ_PER, axis=2)
    k_t = jnp.transpose(k_sl, (0, 2, 1, 3))
    v_t = jnp.transpose(v_sl, (0, 2, 1, 3))
    x2 = x.reshape(ROWS, D_MODEL)

    def body(x_ref, wq_ref, k_ref, v_ref, wo_ref, out_ref,
             ctx_buf, partial_buf, recv_buf, ag_buf,
             rs_send, rs_recv, ag_send, ag_recv):
        my_pos = lax.axis_index("i")

        barrier = pltpu.get_barrier_semaphore()
        for d in range(1, N_DEV):
            peer = lax.rem(my_pos + d, N_DEV)
            pl.semaphore_signal(
                barrier, inc=1,
                device_id=(peer,), device_id_type=pl.DeviceIdType.MESH,
            )

        xb = x_ref[:].astype(jnp.bfloat16)
        wqb = wq_ref[:].astype(jnp.bfloat16)
        q = lax.dot(xb, wqb, preferred_element_type=jnp.float32) * 0.125
        for b in range(B):
            for h in range(HQ_PER):
                qbh = q[b * SQ:(b + 1) * SQ, h * DH:(h + 1) * DH]
                qbh = qbh.astype(jnp.bfloat16)
                kbh = k_ref[b, h].astype(jnp.bfloat16)
                s = lax.dot_general(
                    qbh, kbh, (((1,), (1,)), ((), ())),
                    preferred_element_type=jnp.float32,
                )
                s = s - jnp.max(s, axis=-1, keepdims=True)
                w = jnp.exp(s)
                w = w / jnp.sum(w, axis=-1, keepdims=True)
                vbh = v_ref[b, h].astype(jnp.bfloat16)
                cbh = lax.dot(w.astype(jnp.bfloat16), vbh,
                              preferred_element_type=jnp.float32)
                ctx_buf[b * SQ:(b + 1) * SQ, h * DH:(h + 1) * DH] = cbh
        wob = wo_ref[:].astype(jnp.bfloat16)

        pl.semaphore_wait(barrier, N_DEV - 1)

        rs_ops = []
        for d in range(1, N_DEV):
            t = lax.rem(my_pos + d, N_DEV)
            rows = ctx_buf[pl.ds(t * CHUNK, CHUNK), :].astype(jnp.bfloat16)
            pchunk = lax.dot(rows, wob, preferred_element_type=jnp.float32)
            partial_buf[pl.ds(t, 1)] = pchunk.astype(jnp.bfloat16)[None]
            op = pltpu.make_async_remote_copy(
                src_ref=partial_buf.at[pl.ds(t, 1)],
                dst_ref=recv_buf.at[pl.ds(d, 1)],
                send_sem=rs_send.at[d],
                recv_sem=rs_recv.at[d],
                device_id=(t,),
                device_id_type=pl.DeviceIdType.MESH,
            )
            op.start()
            rs_ops.append(op)
        own_rows = ctx_buf[pl.ds(my_pos * CHUNK, CHUNK), :].astype(jnp.bfloat16)
        acc = lax.dot(own_rows, wob, preferred_element_type=jnp.float32)
        for d in range(1, N_DEV):
            rs_ops[d - 1].wait_recv()
            acc = acc + recv_buf[pl.ds(d, 1)][0].astype(jnp.float32)

        ag_buf[pl.ds(my_pos, 1)] = acc.astype(jnp.bfloat16)[None]
        ag_ops = []
        for d in range(1, N_DEV):
            t = lax.rem(my_pos + d, N_DEV)
            op = pltpu.make_async_remote_copy(
                src_ref=ag_buf.at[pl.ds(my_pos, 1)],
                dst_ref=ag_buf.at[pl.ds(my_pos, 1)],
                send_sem=ag_send.at[d],
                recv_sem=ag_recv.at[d],
                device_id=(t,),
                device_id_type=pl.DeviceIdType.MESH,
            )
            op.start()
            ag_ops.append(op)
        for op in ag_ops:
            op.wait_recv()
        out_ref[:] = ag_buf[:].astype(jnp.float32)

        for op in rs_ops:
            op.wait_send()
        for op in ag_ops:
            op.wait_send()

    out = pl.pallas_call(
        body,
        out_shape=jax.ShapeDtypeStruct((N_DEV, CHUNK, D_MODEL), jnp.float32),
        in_specs=[pl.BlockSpec(memory_space=pltpu.VMEM)] * 5,
        out_specs=pl.BlockSpec(memory_space=pltpu.VMEM),
        scratch_shapes=[
            pltpu.VMEM((ROWS, D_SHARD), jnp.float32),
            pltpu.VMEM((N_DEV, CHUNK, D_MODEL), jnp.bfloat16),
            pltpu.VMEM((N_DEV, CHUNK, D_MODEL), jnp.bfloat16),
            pltpu.VMEM((N_DEV, CHUNK, D_MODEL), jnp.bfloat16),
            pltpu.SemaphoreType.DMA((N_DEV,)),
            pltpu.SemaphoreType.DMA((N_DEV,)),
            pltpu.SemaphoreType.DMA((N_DEV,)),
            pltpu.SemaphoreType.DMA((N_DEV,)),
        ],
        compiler_params=pltpu.CompilerParams(collective_id=0),
    )(x2, Wq, k_t, v_t, Wo)
    return out.reshape(B, SQ, D_MODEL)
